# baseline (speedup 1.0000x reference)
import functools
import math

import jax
import jax.numpy as jnp
import numpy as np
from jax import lax
from jax.experimental import pallas as pl
from jax.experimental.pallas import tpu as pltpu
from jax.experimental.pallas import tpu_sc as plsc

_WS = 32
_N_HASHES = 4
_HALF = 4

_COLOR_R = (0, 46, 167, 100, 191, 220, 0, 10)
_COLOR_G = (160, 141, 0, 62, 30, 87, 166, 91)
_COLOR_B = (177, 239, 174, 191, 75, 46, 0, 196)


def _make_hash_kernel(chg, nH, nW, ws, n_chunks, gsz):
    def _hash_kernel(x_ref, w_ref, code_ref):
        x = x_ref[0].reshape(chg * nH, ws, nW * ws)
        w = w_ref[...]
        v = None
        for jg in range(nW // gsz):
            rhs = jnp.concatenate(
                [x[:, :, (jg * gsz + d) * ws:(jg * gsz + d + 1) * ws]
                 for d in range(gsz)], axis=0)
            pj = jax.lax.dot_general(w[jg], rhs, (((0,), (0,)), ((), ())),
                                     preferred_element_type=jnp.float32)
            v = pj if v is None else v + pj
        codes = []
        for q in range(n_chunks):
            for h in range(_N_HASHES):
                base = q * 16 + h * _HALF
                vs = [v[base + i:base + i + 1] for i in range(_HALF)]
                ab = [jnp.abs(x) for x in vs]
                i01 = jnp.where(ab[1] > ab[0], 1, 0)
                v01 = jnp.where(ab[1] > ab[0], vs[1], vs[0])
                m01 = jnp.maximum(ab[0], ab[1])
                i23 = jnp.where(ab[3] > ab[2], 3, 2)
                v23 = jnp.where(ab[3] > ab[2], vs[3], vs[2])
                m23 = jnp.maximum(ab[2], ab[3])
                sel = m23 > m01
                iabs = jnp.where(sel, i23, i01)
                vsel = jnp.where(sel, v23, v01)
                codes.append(iabs + jnp.where(vsel < 0, 4, 0))
        code = jnp.concatenate(codes, axis=0)
        code_ref[0] = code.reshape(code.shape[0], ws * ws)

    return _hash_kernel


def _color_tables():
    t = np.zeros((12, 16), np.uint32)
    for k, col in enumerate((_COLOR_R, _COLOR_G, _COLOR_B)):
        for h in range(_N_HASHES):
            t[k * 4 + h, :8] = np.uint32(col) << np.uint32(8 * h)
    return jnp.asarray(t.view(np.int32))


def _make_sc_lut(B, M, nW, ws, n_workers):
    n_win = B * M
    tasks = -(-n_win // n_workers)
    T = ws * ws
    n_chunk16 = T // 16

    def _sc_body(codes_hbm, tab_hbm, o_r_hbm, o_g_hbm, o_b_hbm,
                 tab_v, code_v, buf_r, buf_g, buf_b):
        wid = lax.axis_index("s") * 2 + lax.axis_index("c")
        pltpu.sync_copy(tab_hbm, tab_v)

        def one_window(t, carry):
            win = wid * tasks + t

            @pl.when(win < n_win)
            def _():
                b = win // M
                m2 = win % M
                i2 = m2 // nW
                j2 = m2 % nW
                pltpu.sync_copy(codes_hbm.at[b, :, m2], code_v)
                for v in range(n_chunk16):
                    row, co = v // 2, (v % 2) * 16
                    c16 = [code_v[h, pl.ds(v * 16, 16)]
                           for h in range(_N_HASHES)]
                    for k, buf in enumerate((buf_r, buf_g, buf_b)):
                        word = None
                        for h in range(_N_HASHES):
                            g = plsc.load_gather(tab_v.at[k * 4 + h],
                                                 [c16[h]])
                            word = g if word is None else word | g
                        buf[row, pl.ds(co, 16)] = word
                for buf, out in ((buf_r, o_r_hbm), (buf_g, o_g_hbm),
                                 (buf_b, o_b_hbm)):
                    pltpu.sync_copy(
                        buf, out.at[b, pl.ds(i2 * ws, ws),
                                    pl.ds(j2 * ws, ws)])
            return carry

        lax.fori_loop(0, tasks, one_window, 0)

    return _sc_body


def kernel(inp, rotations):
    B, C, H, W = inp.shape
    ws = _WS
    nH, nW = H // ws, W // ws
    M = nH * nW
    T = ws * ws
    R = math.lcm(C, M)
    chg = R // M
    n_chunks = R // C
    n_groups = (B * C) // chg

    x6 = inp.reshape(n_groups, chg, nH, ws, nW * ws)
    w2 = rotations.reshape(C, _N_HASHES * _HALF)
    w3 = jax.scipy.linalg.block_diag(*([w2] * n_chunks))
    gsz = nW
    w3r = (w3.reshape(chg, nH, nW, n_chunks * 16)
              .transpose(2, 0, 1, 3)
              .reshape(nW // gsz, gsz * chg * nH, n_chunks * 16))

    nhr = n_chunks * _N_HASHES
    gpb = n_groups // B
    tc = pl.pallas_call(
        _make_hash_kernel(chg, nH, nW, ws, n_chunks, gsz),
        grid=(gpb,),
        in_specs=[
            pl.BlockSpec((1, chg, nH, ws, nW * ws),
                         lambda n: (n, 0, 0, 0, 0)),
            pl.BlockSpec((nW // gsz, gsz * chg * nH, n_chunks * 16),
                         lambda n: (0, 0, 0)),
        ],
        out_specs=pl.BlockSpec((1, nhr, T), lambda n: (n, 0, 0)),
        out_shape=jax.ShapeDtypeStruct((gpb, nhr, T), jnp.int32),
    )

    n_cores, n_subcores = 2, 16
    n_workers = n_cores * n_subcores
    mesh = plsc.VectorSubcoreMesh(core_axis_name="c", subcore_axis_name="s",
                                  num_cores=n_cores, num_subcores=n_subcores)
    sc = pl.kernel(
        _make_sc_lut(1, M, nW, ws, n_workers),
        out_type=[jax.ShapeDtypeStruct((1, H, W), jnp.int32)
                  for _ in range(3)],
        mesh=mesh,
        scratch_types=[
            pltpu.VMEM((12, 16), jnp.int32),
            pltpu.VMEM((_N_HASHES, T), jnp.int32),
            pltpu.VMEM((ws, ws), jnp.int32),
            pltpu.VMEM((ws, ws), jnp.int32),
            pltpu.VMEM((ws, ws), jnp.int32),
        ],
        compiler_params=pltpu.CompilerParams(use_tc_tiling_on_sc=False,
                                             needs_layout_passes=False),
    )

    tables = _color_tables()
    parts = []
    for b in range(B):
        codes_b = tc(x6[b * gpb:(b + 1) * gpb], w3r)
        parts.append(sc(codes_b.reshape(1, _N_HASHES, M, T), tables))

    def fin(k):
        a = jnp.concatenate([p[k] for p in parts], axis=0)
        return lax.bitcast_convert_type(a, jnp.uint8)

    return fin(0), fin(1), fin(2)


# speedup vs baseline: 1.2970x; 1.2970x over previous
import math

import jax
import jax.numpy as jnp
import numpy as np
from jax import lax
from jax.experimental import pallas as pl
from jax.experimental.pallas import tpu as pltpu
from jax.experimental.pallas import tpu_sc as plsc

_WS = 32
_N_HASHES = 4
_HALF = 4

_COLOR_R = (0, 46, 167, 100, 191, 220, 0, 10)
_COLOR_G = (160, 141, 0, 62, 30, 87, 166, 91)
_COLOR_B = (177, 239, 174, 191, 75, 46, 0, 196)


def _make_hash_kernel(chg, nH, nW, ws, n_chunks, gsz):
    def _hash_kernel(x_ref, w_ref, code_ref):
        x = x_ref[0].reshape(chg * nH, ws, nW * ws)
        w = w_ref[...]
        v = None
        for jg in range(nW // gsz):
            rhs = jnp.concatenate(
                [x[:, :, (jg * gsz + d) * ws:(jg * gsz + d + 1) * ws]
                 for d in range(gsz)], axis=0)
            pj = jax.lax.dot_general(w[jg], rhs, (((0,), (0,)), ((), ())),
                                     preferred_element_type=jnp.float32)
            v = pj if v is None else v + pj
        codes = []
        for q in range(n_chunks):
            for h in range(_N_HASHES):
                base = q * 16 + h * _HALF
                vs = [v[base + i:base + i + 1] for i in range(_HALF)]
                ab = [jnp.abs(x) for x in vs]
                i01 = jnp.where(ab[1] > ab[0], 1, 0)
                v01 = jnp.where(ab[1] > ab[0], vs[1], vs[0])
                m01 = jnp.maximum(ab[0], ab[1])
                i23 = jnp.where(ab[3] > ab[2], 3, 2)
                v23 = jnp.where(ab[3] > ab[2], vs[3], vs[2])
                m23 = jnp.maximum(ab[2], ab[3])
                sel = m23 > m01
                iabs = jnp.where(sel, i23, i01)
                vsel = jnp.where(sel, v23, v01)
                codes.append(iabs + jnp.where(vsel < 0, 4, 0))
        code = jnp.concatenate(codes, axis=0)
        code_ref[0] = code.reshape(code.shape[0], ws * ws)

    return _hash_kernel


def _color_tables():
    t = np.zeros((12, 16), np.uint32)
    for k, col in enumerate((_COLOR_R, _COLOR_G, _COLOR_B)):
        for h in range(_N_HASHES):
            t[k * 4 + h, :8] = np.uint32(col) << np.uint32(8 * h)
    return jnp.asarray(t.view(np.int32))


def _make_sc_lut(B, M, nW, ws, n_workers):
    n_win = B * M
    tasks = -(-n_win // n_workers)
    T = ws * ws
    n_chunk16 = T // 16

    def _sc_body(codes_hbm, tab_hbm, o_r_hbm, o_g_hbm, o_b_hbm,
                 tab_v, code_v, buf_r, buf_g, buf_b):
        wid = lax.axis_index("s") * 2 + lax.axis_index("c")
        pltpu.sync_copy(tab_hbm, tab_v)

        def one_window(t, carry):
            win = wid * tasks + t

            @pl.when(win < n_win)
            def _():
                b = win // M
                m2 = win % M
                i2 = m2 // nW
                j2 = m2 % nW
                pltpu.sync_copy(codes_hbm.at[b, :, m2], code_v)
                for v in range(n_chunk16):
                    row, co = v // 2, (v % 2) * 16
                    c16 = [code_v[h, pl.ds(v * 16, 16)]
                           for h in range(_N_HASHES)]
                    for k, buf in enumerate((buf_r, buf_g, buf_b)):
                        word = None
                        for h in range(_N_HASHES):
                            g = plsc.load_gather(tab_v.at[k * 4 + h],
                                                 [c16[h]])
                            word = g if word is None else word | g
                        buf[row, pl.ds(co, 16)] = word
                for buf, out in ((buf_r, o_r_hbm), (buf_g, o_g_hbm),
                                 (buf_b, o_b_hbm)):
                    pltpu.sync_copy(
                        buf, out.at[b, pl.ds(i2 * ws, ws),
                                    pl.ds(j2 * ws, ws)])
            return carry

        lax.fori_loop(0, tasks, one_window, 0)

    return _sc_body


def kernel(inp, rotations):
    B, C, H, W = inp.shape
    ws = _WS
    nH, nW = H // ws, W // ws
    M = nH * nW
    T = ws * ws
    R = math.lcm(C, M)
    chg = R // M
    n_chunks = R // C
    n_groups = (B * C) // chg

    x6 = inp.reshape(n_groups, chg, nH, ws, nW * ws)
    w2 = rotations.reshape(C, _N_HASHES * _HALF)
    w3 = jax.scipy.linalg.block_diag(*([w2] * n_chunks))
    gsz = nW
    w3r = (w3.reshape(chg, nH, nW, n_chunks * 16)
              .transpose(2, 0, 1, 3)
              .reshape(nW // gsz, gsz * chg * nH, n_chunks * 16))

    nhr = n_chunks * _N_HASHES
    codes = pl.pallas_call(
        _make_hash_kernel(chg, nH, nW, ws, n_chunks, gsz),
        grid=(n_groups,),
        in_specs=[
            pl.BlockSpec((1, chg, nH, ws, nW * ws),
                         lambda n: (n, 0, 0, 0, 0)),
            pl.BlockSpec((nW // gsz, gsz * chg * nH, n_chunks * 16),
                         lambda n: (0, 0, 0)),
        ],
        out_specs=pl.BlockSpec((1, nhr, T), lambda n: (n, 0, 0)),
        out_shape=jax.ShapeDtypeStruct((n_groups, nhr, T), jnp.int32),
    )(x6, w3r)

    codes_flat = codes.reshape(B, _N_HASHES, M, T)

    n_cores, n_subcores = 2, 16
    n_workers = n_cores * n_subcores
    mesh = plsc.VectorSubcoreMesh(core_axis_name="c", subcore_axis_name="s",
                                  num_cores=n_cores, num_subcores=n_subcores)
    sc = pl.kernel(
        _make_sc_lut(B, M, nW, ws, n_workers),
        out_type=[jax.ShapeDtypeStruct((B, H, W), jnp.int32)
                  for _ in range(3)],
        mesh=mesh,
        scratch_types=[
            pltpu.VMEM((12, 16), jnp.int32),
            pltpu.VMEM((_N_HASHES, T), jnp.int32),
            pltpu.VMEM((ws, ws), jnp.int32),
            pltpu.VMEM((ws, ws), jnp.int32),
            pltpu.VMEM((ws, ws), jnp.int32),
        ],
        compiler_params=pltpu.CompilerParams(use_tc_tiling_on_sc=False,
                                             needs_layout_passes=False),
    )
    r32, g32, b32 = sc(codes_flat, _color_tables())

    def fin(a):
        return lax.bitcast_convert_type(a, jnp.uint8)

    return fin(r32), fin(g32), fin(b32)
